# baseline (speedup 1.0000x reference)
import functools

import jax
import jax.numpy as jnp
from jax import lax
from jax.experimental import pallas as pl
from jax.experimental.pallas import tpu as pltpu
from jax.experimental.pallas import tpu_sc as plsc

B = 16384
D = 32
NC = 2
NS = 16
NW = NC * NS
R = B // NW
CH = 128
NCH = R // CH
NJ = 1008
FIRE = 8


def _proj_body(ba_ref, ta_ref, wpa_ref, wa_ref, gt_ref):
    p_t = lax.dot_general(wpa_ref[...], ta_ref[...], (((1,), (1,)), ((), ())),
                          preferred_element_type=jnp.float32)
    logit = lax.dot_general(wa_ref[...], p_t, (((1,), (0,)), ((), ())),
                            preferred_element_type=jnp.float32) + ba_ref[0]
    gt_ref[...] = p_t / (1.0 + jnp.exp(-logit))


def _project_table(ta, wpa, w_attn, b_attn):
    return pl.pallas_call(
        _proj_body,
        grid=(),
        in_specs=[
            pl.BlockSpec(memory_space=pltpu.SMEM),
            pl.BlockSpec((NJ, 17), lambda: (0, 0)),
            pl.BlockSpec((D, 17), lambda: (0, 0)),
            pl.BlockSpec((1, D), lambda: (0, 0)),
        ],
        out_specs=pl.BlockSpec((D, NJ), lambda: (0, 0)),
        out_shape=jax.ShapeDtypeStruct((D, NJ), jnp.float32),
    )(b_attn, ta, wpa, w_attn)


def _sc_body(uid, iid, jid, utt, itt, gtt, out,
             uidx, iidx, jidx, mbuf, ucols, icols, gcols, outv, sem):
    wid = lax.axis_index("s") * NC + lax.axis_index("c")
    base = wid * R

    for c in range(NCH):
        pltpu.sync_copy(uid.at[pl.ds(base + c * CH, CH)], uidx.at[c])
        pltpu.sync_copy(iid.at[pl.ds(base + c * CH, CH)], iidx.at[c])
        pltpu.sync_copy(jid.at[pl.ds(base + c * CH, CH)], jidx.at[c])

    onef = jnp.full((16,), 1.0, jnp.float32)
    zerof = jnp.full((16,), 0.0, jnp.float32)
    zi = jnp.full((16,), 0, jnp.int32)
    for c in range(NCH):
        for s in range(CH // 16):
            jv = jidx[c, pl.ds(s * 16, 16)]
            ok = jv >= 0
            jidx[c, pl.ds(s * 16, 16)] = jnp.where(ok, jv, zi)
            mbuf[pl.ds(c * CH + s * 16, 16)] = jnp.where(ok, onef, zerof)

    jobs = []
    for k in range(D):
        for c in range(NCH):
            sl = pl.ds(c * CH, CH)
            jobs.append((utt, uidx, k, c, ucols))
            jobs.append((itt, iidx, k, c, icols))
            jobs.append((gtt, jidx, k, c, gcols))
    pend = []
    for (tab, idx, k, c, dst) in jobs:
        pend.append(pltpu.async_copy(
            tab.at[k].at[idx.at[c]], dst.at[k, pl.ds(c * CH, CH)], sem))
        if len(pend) == FIRE:
            for cp in pend:
                cp.wait()
            pend = []
    for cp in pend:
        cp.wait()

    def group(g, carry):
        sl = pl.ds(g * 16, 16)
        m = mbuf[sl]
        acc = jnp.full((16,), 0.0, jnp.float32)
        for k in range(D):
            acc = acc + (ucols[k, sl] + m * gcols[k, sl]) * icols[k, sl]
        outv[sl] = acc
        return carry

    lax.fori_loop(0, R // 16, group, 0)

    pltpu.sync_copy(outv, out.at[pl.ds(base, R)])


_sc_call = functools.partial(
    pl.kernel,
    out_type=jax.ShapeDtypeStruct((B,), jnp.float32),
    mesh=plsc.VectorSubcoreMesh(core_axis_name="c", subcore_axis_name="s",
                                num_cores=NC, num_subcores=NS),
    compiler_params=pltpu.CompilerParams(use_tc_tiling_on_sc=False),
    scratch_types=[
        pltpu.VMEM((NCH, CH), jnp.int32),
        pltpu.VMEM((NCH, CH), jnp.int32),
        pltpu.VMEM((NCH, CH), jnp.int32),
        pltpu.VMEM((R,), jnp.float32),
        pltpu.VMEM((D, R), jnp.float32),
        pltpu.VMEM((D, R), jnp.float32),
        pltpu.VMEM((D, R), jnp.float32),
        pltpu.VMEM((R,), jnp.float32),
        pltpu.SemaphoreType.DMA,
    ],
)


def kernel(user_id, item_id, jobcat_id, user_table, item_table,
           jobcat_table, W_proj, b_proj, W_attn, b_attn):
    n = jobcat_table.shape[0]
    t_pad = jnp.pad(jobcat_table, ((0, NJ - n), (0, 0)))
    ta = jnp.concatenate([t_pad, jnp.ones((NJ, 1), jnp.float32)], axis=1)
    wpa = jnp.concatenate([W_proj, b_proj.reshape(D, 1)], axis=1)
    gt_t = _project_table(ta, wpa, W_attn, b_attn)
    return _sc_call(_sc_body)(user_id, item_id, jobcat_id,
                              user_table.T, item_table.T, gt_t)


# speedup vs baseline: 1.8280x; 1.8280x over previous
import functools

import jax
import jax.numpy as jnp
from jax import lax
from jax.experimental import pallas as pl
from jax.experimental.pallas import tpu as pltpu
from jax.experimental.pallas import tpu_sc as plsc

B = 16384
D = 32
NC = 2
NS = 16
NW = NC * NS
R = B // NW
CH = 128
NCH = R // CH
BLK = 2048
NJ = 1008
NG = NJ * D


def _proj_body(ba_ref, t_ref, wp_ref, bp_ref, wa_ref, g_ref):
    t = t_ref[...]
    p = lax.dot_general(t, wp_ref[...], (((1,), (1,)), ((), ())),
                        preferred_element_type=jnp.float32) + bp_ref[...]
    q = p * wa_ref[...]
    ones = jnp.ones((D, D), jnp.float32)
    logit = lax.dot_general(q, ones, (((1,), (0,)), ((), ())),
                            preferred_element_type=jnp.float32) + ba_ref[0]
    g_ref[...] = p / (1.0 + jnp.exp(-logit))


def _project_table(t_pad, w_proj, b_proj, w_attn, b_attn):
    return pl.pallas_call(
        _proj_body,
        grid=(),
        in_specs=[
            pl.BlockSpec(memory_space=pltpu.SMEM),
            pl.BlockSpec((NJ, 16), lambda: (0, 0)),
            pl.BlockSpec((D, 16), lambda: (0, 0)),
            pl.BlockSpec((1, D), lambda: (0, 0)),
            pl.BlockSpec((1, D), lambda: (0, 0)),
        ],
        out_specs=pl.BlockSpec((NJ, D), lambda: (0, 0)),
        out_shape=jax.ShapeDtypeStruct((NJ, D), jnp.float32),
    )(b_attn, t_pad, w_proj, b_proj.reshape(1, D), w_attn)


def _sc_body(uid, iid, jid, ut4, it4, g1, prod_out,
             uidx, iidx, uoff, ioff, goff, mbuf, gvm, upk, ipk, prodv, sem):
    wid = lax.axis_index("s") * NC + lax.axis_index("c")
    base = wid * R

    pltpu.sync_copy(g1, gvm)

    for c in range(NCH):
        pltpu.sync_copy(uid.at[pl.ds(base + c * CH, CH)], uidx.at[c])
        pltpu.sync_copy(iid.at[pl.ds(base + c * CH, CH)], iidx.at[c])
        pltpu.sync_copy(jid.at[pl.ds(base + c * CH, CH)], goff.at[pl.ds(c * CH, CH)])

    onef = jnp.full((16,), 1.0, jnp.float32)
    zerof = jnp.full((16,), 0.0, jnp.float32)
    zi = jnp.full((16,), 0, jnp.int32)
    for c in range(NCH):
        for s in range(CH // 16):
            a = c * CH + s * 16
            uv = uidx[c, pl.ds(s * 16, 16)]
            uoff[pl.ds(a, 16)] = (uv & 3) << 5
            uidx[c, pl.ds(s * 16, 16)] = uv >> 2
            iv = iidx[c, pl.ds(s * 16, 16)]
            ioff[pl.ds(a, 16)] = (iv & 3) << 5
            iidx[c, pl.ds(s * 16, 16)] = iv >> 2
            jv = goff[pl.ds(a, 16)]
            ok = jv >= 0
            goff[pl.ds(a, 16)] = jnp.where(ok, jv, zi) << 5
            mbuf[pl.ds(a, 16)] = jnp.where(ok, onef, zerof)

    for c in range(NCH):
        cpu = pltpu.async_copy(ut4.at[uidx.at[c]], upk, sem)
        cpi = pltpu.async_copy(it4.at[iidx.at[c]], ipk, sem)
        cpu.wait()
        cpi.wait()

        def grp(g, carry, c=c):
            ab = c * CH + g * 16
            mv = mbuf[pl.ds(ab, 16)]
            ouv = uoff[pl.ds(ab, 16)]
            oiv = ioff[pl.ds(ab, 16)]
            ogv = goff[pl.ds(ab, 16)]
            for t in range(16):
                r = g * 16 + t
                ou = ouv[t]
                oi = oiv[t]
                og = ogv[t]
                m = mv[t]
                u0 = upk[r, pl.ds(ou, 16)]
                u1 = upk[r, pl.ds(ou + 16, 16)]
                i0 = ipk[r, pl.ds(oi, 16)]
                i1 = ipk[r, pl.ds(oi + 16, 16)]
                g0 = gvm[pl.ds(og, 16)]
                g1v = gvm[pl.ds(og + 16, 16)]
                pr = (ab + t) * D
                prodv[pl.ds(pr, 16)] = (u0 + m * g0) * i0
                prodv[pl.ds(pr + 16, 16)] = (u1 + m * g1v) * i1
            return carry

        lax.fori_loop(0, CH // 16, grp, 0)

    pltpu.sync_copy(prodv, prod_out.at[pl.ds(base * D, R * D)])


_sc_gather = functools.partial(
    pl.kernel,
    out_type=jax.ShapeDtypeStruct((B * D,), jnp.float32),
    mesh=plsc.VectorSubcoreMesh(core_axis_name="c", subcore_axis_name="s",
                                num_cores=NC, num_subcores=NS),
    scratch_types=[
        pltpu.VMEM((NCH, CH), jnp.int32),
        pltpu.VMEM((NCH, CH), jnp.int32),
        pltpu.VMEM((R,), jnp.int32),
        pltpu.VMEM((R,), jnp.int32),
        pltpu.VMEM((R,), jnp.int32),
        pltpu.VMEM((R,), jnp.float32),
        pltpu.VMEM((NG,), jnp.float32),
        pltpu.VMEM((CH, 128), jnp.float32),
        pltpu.VMEM((CH, 128), jnp.float32),
        pltpu.VMEM((R * D,), jnp.float32),
        pltpu.SemaphoreType.DMA,
    ],
)


def _pack_body(a_ref, t_ref, out_ref):
    a = a_ref[...]
    tb = t_ref[...].astype(jnp.bfloat16)
    cols = []
    for q in range(4):
        cq = lax.dot_general(a[:, q * 512:(q + 1) * 512], tb,
                             (((1,), (1,)), ((), ())),
                             preferred_element_type=jnp.float32)
        cols.append(cq)
    out_ref[...] = jnp.concatenate(cols, axis=1)


def _pack_table(table_t, a_sel):
    n = table_t.shape[1]
    grid = (n + 511) // 512
    return pl.pallas_call(
        _pack_body,
        grid=(grid,),
        in_specs=[
            pl.BlockSpec((128, 2048), lambda j: (0, 0)),
            pl.BlockSpec((D, 512), lambda j: (0, j)),
        ],
        out_specs=pl.BlockSpec((128, 128), lambda j: (j, 0)),
        out_shape=jax.ShapeDtypeStruct((grid * 128, 128), jnp.float32),
    )(a_sel, table_t)


def _make_selector():
    pp = jnp.arange(128)
    a = jnp.zeros((128, 2048), jnp.bfloat16)
    for q in range(4):
        a = a.at[pp, q * 512 + 4 * pp + q].set(1)
    return a


def _reduce_body(p_ref, o_ref):
    o_ref[...] = jnp.sum(p_ref[...], axis=1, keepdims=True)


def _row_reduce(prod2d):
    return pl.pallas_call(
        _reduce_body,
        grid=(B // BLK,),
        in_specs=[pl.BlockSpec((BLK, D), lambda i: (i, 0))],
        out_specs=pl.BlockSpec((BLK, 1), lambda i: (i, 0)),
        out_shape=jax.ShapeDtypeStruct((B, 1), jnp.float32),
    )(prod2d)


def kernel(user_id, item_id, jobcat_id, user_table, item_table,
           jobcat_table, W_proj, b_proj, W_attn, b_attn):
    n = jobcat_table.shape[0]
    t_pad = jnp.pad(jobcat_table, ((0, NJ - n), (0, 0)))
    g_table = _project_table(t_pad, W_proj, b_proj, W_attn, b_attn)
    a_sel = _make_selector()
    ut4 = _pack_table(user_table.T, a_sel)
    it4 = _pack_table(item_table.T, a_sel)
    prod = _sc_gather(_sc_body)(
        user_id, item_id, jobcat_id, ut4, it4, g_table.reshape(NG))
    out2d = _row_reduce(prod.reshape(B, D))
    return out2d.reshape(B)


# speedup vs baseline: 7.8895x; 4.3160x over previous
import functools

import jax
import jax.numpy as jnp
from jax import lax
from jax.experimental import pallas as pl
from jax.experimental.pallas import tpu as pltpu
from jax.experimental.pallas import tpu_sc as plsc

B = 16384
D = 32
NC = 2
NS = 16
NW = NC * NS
R = B // NW
CH = 128
NCH = R // CH
BLK = 2048
NJ = 1008
NG = NJ * D


def _proj_body(ba_ref, t_ref, wp_ref, bp_ref, wa_ref, g_ref):
    t = t_ref[...]
    p = lax.dot_general(t, wp_ref[...], (((1,), (1,)), ((), ())),
                        preferred_element_type=jnp.float32) + bp_ref[...]
    q = p * wa_ref[...]
    ones = jnp.ones((D, D), jnp.float32)
    logit = lax.dot_general(q, ones, (((1,), (0,)), ((), ())),
                            preferred_element_type=jnp.float32) + ba_ref[0]
    g_ref[...] = p / (1.0 + jnp.exp(-logit))


def _project_table(t_pad, w_proj, b_proj, w_attn, b_attn):
    return pl.pallas_call(
        _proj_body,
        grid=(),
        in_specs=[
            pl.BlockSpec(memory_space=pltpu.SMEM),
            pl.BlockSpec((NJ, 16), lambda: (0, 0)),
            pl.BlockSpec((D, 16), lambda: (0, 0)),
            pl.BlockSpec((1, D), lambda: (0, 0)),
            pl.BlockSpec((1, D), lambda: (0, 0)),
        ],
        out_specs=pl.BlockSpec((NJ, D), lambda: (0, 0)),
        out_shape=jax.ShapeDtypeStruct((NJ, D), jnp.float32),
    )(b_attn, t_pad, w_proj, b_proj.reshape(1, D), w_attn)


def _sc_body(uid, iid, jid, ut4, it4, g1, prod_out,
             uidx, iidx, uoff, ioff, goff, mbuf, gvm, upk, ipk, prodv, sem):
    wid = lax.axis_index("s") * NC + lax.axis_index("c")
    base = wid * R

    pltpu.sync_copy(g1, gvm)

    for c in range(NCH):
        pltpu.sync_copy(uid.at[pl.ds(base + c * CH, CH)], uidx.at[c])
        pltpu.sync_copy(iid.at[pl.ds(base + c * CH, CH)], iidx.at[c])
        pltpu.sync_copy(jid.at[pl.ds(base + c * CH, CH)], goff.at[pl.ds(c * CH, CH)])

    onef = jnp.full((16,), 1.0, jnp.float32)
    zerof = jnp.full((16,), 0.0, jnp.float32)
    zi = jnp.full((16,), 0, jnp.int32)
    for c in range(NCH):
        for s in range(CH // 16):
            a = c * CH + s * 16
            uv = uidx[c, pl.ds(s * 16, 16)]
            uoff[pl.ds(a, 16)] = ((uv >> 12) & 3) << 5
            uidx[c, pl.ds(s * 16, 16)] = ((uv >> 14) << 12) | (uv & 4095)
            iv = iidx[c, pl.ds(s * 16, 16)]
            ioff[pl.ds(a, 16)] = ((iv >> 12) & 3) << 5
            iidx[c, pl.ds(s * 16, 16)] = ((iv >> 14) << 12) | (iv & 4095)
            jv = goff[pl.ds(a, 16)]
            ok = jv >= 0
            goff[pl.ds(a, 16)] = jnp.where(ok, jv, zi) << 5
            mbuf[pl.ds(a, 16)] = jnp.where(ok, onef, zerof)

    for c in range(NCH):
        cpu = pltpu.async_copy(ut4.at[uidx.at[c]], upk, sem)
        cpi = pltpu.async_copy(it4.at[iidx.at[c]], ipk, sem)
        cpu.wait()
        cpi.wait()

        def grp(g, carry, c=c):
            ab = c * CH + g * 16
            mv = mbuf[pl.ds(ab, 16)]
            ouv = uoff[pl.ds(ab, 16)]
            oiv = ioff[pl.ds(ab, 16)]
            ogv = goff[pl.ds(ab, 16)]
            for t in range(16):
                r = g * 16 + t
                ou = ouv[t]
                oi = oiv[t]
                og = ogv[t]
                m = mv[t]
                u0 = upk[r, pl.ds(ou, 16)]
                u1 = upk[r, pl.ds(ou + 16, 16)]
                i0 = ipk[r, pl.ds(oi, 16)]
                i1 = ipk[r, pl.ds(oi + 16, 16)]
                g0 = gvm[pl.ds(og, 16)]
                g1v = gvm[pl.ds(og + 16, 16)]
                pr = (ab + t) * D
                prodv[pl.ds(pr, 16)] = (u0 + m * g0) * i0
                prodv[pl.ds(pr + 16, 16)] = (u1 + m * g1v) * i1
            return carry

        lax.fori_loop(0, CH // 16, grp, 0)

    pltpu.sync_copy(prodv, prod_out.at[pl.ds(base * D, R * D)])


_sc_gather = functools.partial(
    pl.kernel,
    out_type=jax.ShapeDtypeStruct((B * D,), jnp.float32),
    mesh=plsc.VectorSubcoreMesh(core_axis_name="c", subcore_axis_name="s",
                                num_cores=NC, num_subcores=NS),
    scratch_types=[
        pltpu.VMEM((NCH, CH), jnp.int32),
        pltpu.VMEM((NCH, CH), jnp.int32),
        pltpu.VMEM((R,), jnp.int32),
        pltpu.VMEM((R,), jnp.int32),
        pltpu.VMEM((R,), jnp.int32),
        pltpu.VMEM((R,), jnp.float32),
        pltpu.VMEM((NG,), jnp.float32),
        pltpu.VMEM((CH, 128), jnp.float32),
        pltpu.VMEM((CH, 128), jnp.float32),
        pltpu.VMEM((R * D,), jnp.float32),
        pltpu.SemaphoreType.DMA,
    ],
)


PCOL = 16384


def _pack_body(t_ref, out_ref):
    tt = t_ref[...].T
    q4 = PCOL // 4
    out_ref[...] = jnp.concatenate(
        [tt[q * q4:(q + 1) * q4] for q in range(4)], axis=1)


def _pack_table(table_t):
    n = table_t.shape[1]
    grid = (n + PCOL - 1) // PCOL
    return pl.pallas_call(
        _pack_body,
        grid=(grid,),
        in_specs=[pl.BlockSpec((D, PCOL), lambda j: (0, j))],
        out_specs=pl.BlockSpec((PCOL // 4, 128), lambda j: (j, 0)),
        out_shape=jax.ShapeDtypeStruct((grid * PCOL // 4, 128), jnp.float32),
    )(table_t)


def _reduce_body(p_ref, o_ref):
    o_ref[...] = jnp.sum(p_ref[...], axis=1, keepdims=True)


def _row_reduce(prod2d):
    return pl.pallas_call(
        _reduce_body,
        grid=(B // BLK,),
        in_specs=[pl.BlockSpec((BLK, D), lambda i: (i, 0))],
        out_specs=pl.BlockSpec((BLK, 1), lambda i: (i, 0)),
        out_shape=jax.ShapeDtypeStruct((B, 1), jnp.float32),
    )(prod2d)


def kernel(user_id, item_id, jobcat_id, user_table, item_table,
           jobcat_table, W_proj, b_proj, W_attn, b_attn):
    n = jobcat_table.shape[0]
    t_pad = jnp.pad(jobcat_table, ((0, NJ - n), (0, 0)))
    g_table = _project_table(t_pad, W_proj, b_proj, W_attn, b_attn)
    ut4 = _pack_table(user_table.T)
    it4 = _pack_table(item_table.T)
    prod = _sc_gather(_sc_body)(
        user_id, item_id, jobcat_id, ut4, it4, g_table.reshape(NG))
    out2d = _row_reduce(prod.reshape(B, D))
    return out2d.reshape(B)
